# Initial kernel scaffold; baseline (speedup 1.0000x reference)
import jax
import jax.numpy as jnp
from jax.experimental import pallas as pl


def kernel(x, adj, W1, b1, W2, b2):
    raise NotImplementedError("write your pallas kernel here")


# speedup vs baseline: 23.7977x; 23.7977x over previous
import functools

import jax
import jax.numpy as jnp
from jax import lax
from jax.experimental import pallas as pl
from jax.experimental.pallas import tpu as pltpu
from jax.experimental.pallas import tpu_sc as plsc

N = 10000
E = 320000
NF = 128
NH = 40
NC = 7
D2 = 8

NWORK = 32
CHUNK = 128
NCHUNK = 80
E_PAD = NWORK * NCHUNK * CHUNK
N_PAD = 10240
TRASH = N
RPS = N_PAD // 16


def _mesh():
    return plsc.VectorSubcoreMesh(core_axis_name="c", subcore_axis_name="s")


_SC_PARAMS = pltpu.CompilerParams(use_tc_tiling_on_sc=False)


@functools.partial(
    pl.kernel,
    out_type=jax.ShapeDtypeStruct((2, N_PAD), jnp.float32),
    mesh=_mesh(),
    compiler_params=_SC_PARAMS,
    scratch_types=[
        pltpu.VMEM((NCHUNK, CHUNK), jnp.int32),
        pltpu.VMEM((CHUNK,), jnp.float32),
        pltpu.VMEM_SHARED((N_PAD,), jnp.float32),
    ],
)
def _deg_kernel(dst_hbm, zeros_hbm, ones_hbm, out_hbm, dst_v, ones_v, acc_sh):
    c = lax.axis_index("c")
    s = lax.axis_index("s")
    wid = c * 16 + s
    pltpu.sync_copy(dst_hbm.at[wid], dst_v)
    pltpu.sync_copy(ones_hbm, ones_v)
    sl = pl.ds(s * RPS, RPS)
    pltpu.sync_copy(zeros_hbm.at[sl], acc_sh.at[sl])
    plsc.subcore_barrier()

    def chunk(j, carry):
        pltpu.sync_copy(ones_v, acc_sh.at[dst_v.at[j]], add=True)
        return carry

    lax.fori_loop(0, NCHUNK, chunk, 0)
    plsc.subcore_barrier()
    pltpu.sync_copy(acc_sh.at[sl], out_hbm.at[c, sl])


def _make_spmm(d):
    @functools.partial(
        pl.kernel,
        out_type=jax.ShapeDtypeStruct((2, N_PAD, d), jnp.float32),
        mesh=_mesh(),
        compiler_params=_SC_PARAMS,
        scratch_types=[
            pltpu.VMEM((NCHUNK, CHUNK), jnp.int32),
            pltpu.VMEM((NCHUNK, CHUNK), jnp.int32),
            pltpu.VMEM((CHUNK, d), jnp.float32),
            pltpu.VMEM_SHARED((N_PAD, d), jnp.float32),
        ],
    )
    def spmm(src_hbm, dst_hbm, y_hbm, zeros_hbm, out_hbm,
             src_v, dst_v, rows_v, acc_sh):
        c = lax.axis_index("c")
        s = lax.axis_index("s")
        wid = c * 16 + s
        pltpu.sync_copy(src_hbm.at[wid], src_v)
        pltpu.sync_copy(dst_hbm.at[wid], dst_v)
        sl = pl.ds(s * RPS, RPS)
        pltpu.sync_copy(zeros_hbm.at[sl], acc_sh.at[sl])
        plsc.subcore_barrier()

        def chunk(j, carry):
            pltpu.sync_copy(y_hbm.at[src_v.at[j]], rows_v)
            pltpu.sync_copy(rows_v, acc_sh.at[dst_v.at[j]], add=True)
            return carry

        lax.fori_loop(0, NCHUNK, chunk, 0)
        plsc.subcore_barrier()
        pltpu.sync_copy(acc_sh.at[sl], out_hbm.at[c, sl])

    return spmm


_spmm40 = _make_spmm(NH)
_spmm8 = _make_spmm(D2)


def _mm1_body(x_ref, w_ref, o_ref):
    o_ref[...] = jnp.dot(x_ref[...], w_ref[...],
                         preferred_element_type=jnp.float32)


def _scale_body(xw_ref, dego_ref, y_ref, dinv_ref):
    deg = dego_ref[0, :N] + dego_ref[1, :N] + 1.0
    dinv = lax.rsqrt(deg)[:, None]
    dinv_ref[...] = dinv
    y_ref[...] = xw_ref[...] * dinv


def _dense2_body(za_ref, zb_ref, y1_ref, dinv_ref, b1_ref, w2_ref, y2_ref):
    dinv = dinv_ref[...]
    h = dinv * (za_ref[...] + zb_ref[...] + y1_ref[...]) + b1_ref[...][None, :]
    h = jnp.maximum(h, 0.0)
    y2 = jnp.dot(h, w2_ref[...], preferred_element_type=jnp.float32) * dinv
    y2_ref[...] = jnp.concatenate(
        [y2, jnp.zeros((N, D2 - NC), jnp.float32)], axis=1)


def _final_body(za_ref, zb_ref, y2_ref, dinv_ref, b2_ref, o_ref):
    t = dinv_ref[...] * (za_ref[...] + zb_ref[...] + y2_ref[...])
    logits = t[:, :NC] + b2_ref[...][None, :]
    m = jnp.max(logits, axis=1, keepdims=True)
    lse = jnp.log(jnp.sum(jnp.exp(logits - m), axis=1, keepdims=True)) + m
    o_ref[...] = logits - lse


def _tc(body, out_shape, *args):
    return pl.pallas_call(body, out_shape=out_shape)(*args)


def kernel(x, adj, W1, b1, W2, b2):
    src = adj[0].astype(jnp.int32)
    dst = adj[1].astype(jnp.int32)
    pad = E_PAD - E
    src3 = jnp.concatenate([src, jnp.zeros((pad,), jnp.int32)])
    src3 = src3.reshape(NWORK, NCHUNK, CHUNK)
    dst3 = jnp.concatenate([dst, jnp.full((pad,), TRASH, jnp.int32)])
    dst3 = dst3.reshape(NWORK, NCHUNK, CHUNK)

    zeros1 = jnp.zeros((N_PAD,), jnp.float32)
    zeros40 = jnp.zeros((N_PAD, NH), jnp.float32)
    zeros8 = jnp.zeros((N_PAD, D2), jnp.float32)
    ones = jnp.ones((CHUNK,), jnp.float32)

    dego = _deg_kernel(dst3, zeros1, ones)
    xw = _tc(_mm1_body, jax.ShapeDtypeStruct((N, NH), jnp.float32), x, W1)

    y1, dinv = pl.pallas_call(
        _scale_body,
        out_shape=[jax.ShapeDtypeStruct((N, NH), jnp.float32),
                   jax.ShapeDtypeStruct((N, 1), jnp.float32)],
    )(xw, dego)

    z1 = _spmm40(src3, dst3, y1, zeros40)
    y2 = _tc(_dense2_body, jax.ShapeDtypeStruct((N, D2), jnp.float32),
             z1[0, :N], z1[1, :N], y1, dinv, b1, W2)

    z2 = _spmm8(src3, dst3, y2, zeros8)
    out = _tc(_final_body, jax.ShapeDtypeStruct((N, NC), jnp.float32),
              z2[0, :N], z2[1, :N], y2, dinv, b2)
    return out


# speedup vs baseline: 30.1810x; 1.2682x over previous
import functools

import jax
import jax.numpy as jnp
from jax import lax
from jax.experimental import pallas as pl
from jax.experimental.pallas import tpu as pltpu
from jax.experimental.pallas import tpu_sc as plsc

N = 10000
E = 320000
NF = 128
NH = 40
NC = 7
D2 = 8

NWORK = 32
CHUNK = 128
NCHUNK = 80
E_PAD = NWORK * NCHUNK * CHUNK
N_PAD = 10240
TRASH = N
RPS = N_PAD // 16


def _mesh():
    return plsc.VectorSubcoreMesh(core_axis_name="c", subcore_axis_name="s")


_SC_PARAMS = pltpu.CompilerParams(use_tc_tiling_on_sc=False)


@functools.partial(
    pl.kernel,
    out_type=jax.ShapeDtypeStruct((2, N_PAD), jnp.float32),
    mesh=_mesh(),
    compiler_params=_SC_PARAMS,
    scratch_types=[
        pltpu.VMEM((NCHUNK, CHUNK), jnp.int32),
        pltpu.VMEM((CHUNK,), jnp.float32),
        pltpu.VMEM_SHARED((N_PAD,), jnp.float32),
    ],
)
def _deg_kernel(dst_hbm, zeros_hbm, ones_hbm, out_hbm, dst_v, ones_v, acc_sh):
    c = lax.axis_index("c")
    s = lax.axis_index("s")
    wid = c * 16 + s
    pltpu.sync_copy(dst_hbm.at[wid], dst_v)
    pltpu.sync_copy(ones_hbm, ones_v)
    sl = pl.ds(s * RPS, RPS)
    pltpu.sync_copy(zeros_hbm.at[sl], acc_sh.at[sl])
    plsc.subcore_barrier()

    def chunk(j, carry):
        pltpu.sync_copy(ones_v, acc_sh.at[dst_v.at[j]], add=True)
        return carry

    lax.fori_loop(0, NCHUNK, chunk, 0)
    plsc.subcore_barrier()
    pltpu.sync_copy(acc_sh.at[sl], out_hbm.at[c, sl])


NBUF = 4


def _make_spmm(d):
    @functools.partial(
        pl.kernel,
        out_type=jax.ShapeDtypeStruct((2, N_PAD, d), jnp.float32),
        mesh=_mesh(),
        compiler_params=_SC_PARAMS,
        scratch_types=[
            pltpu.VMEM((NCHUNK, CHUNK), jnp.int32),
            pltpu.VMEM((NCHUNK, CHUNK), jnp.int32),
            pltpu.VMEM((NBUF, CHUNK, d), jnp.float32),
            pltpu.VMEM_SHARED((N_PAD, d), jnp.float32),
            pltpu.SemaphoreType.DMA((NBUF,)),
        ],
    )
    def spmm(src_hbm, dst_hbm, y_hbm, zeros_hbm, out_hbm,
             src_v, dst_v, rows_v, acc_sh, gsem):
        c = lax.axis_index("c")
        s = lax.axis_index("s")
        wid = c * 16 + s
        pltpu.sync_copy(src_hbm.at[wid], src_v)
        pltpu.sync_copy(dst_hbm.at[wid], dst_v)
        sl = pl.ds(s * RPS, RPS)
        pltpu.sync_copy(zeros_hbm.at[sl], acc_sh.at[sl])
        plsc.subcore_barrier()

        for b in range(NBUF):
            pltpu.make_async_copy(
                y_hbm.at[src_v.at[b]], rows_v.at[b], gsem.at[b]).start()

        def group(i, carry):
            base = i * NBUF
            for b in range(NBUF):
                j = base + b
                pltpu.make_async_copy(
                    y_hbm.at[src_v.at[j]], rows_v.at[b], gsem.at[b]).wait()
                pltpu.sync_copy(rows_v.at[b], acc_sh.at[dst_v.at[j]],
                                add=True)
                nxt = j + NBUF

                @pl.when(nxt < NCHUNK)
                def _():
                    pltpu.make_async_copy(
                        y_hbm.at[src_v.at[nxt]], rows_v.at[b],
                        gsem.at[b]).start()
            return carry

        lax.fori_loop(0, NCHUNK // NBUF, group, 0)
        plsc.subcore_barrier()
        pltpu.sync_copy(acc_sh.at[sl], out_hbm.at[c, sl])

    return spmm


_spmm40 = _make_spmm(NH)
_spmm8 = _make_spmm(D2)


def _mm1_body(x_ref, w_ref, o_ref):
    o_ref[...] = jnp.dot(x_ref[...], w_ref[...],
                         preferred_element_type=jnp.float32)


def _scale_body(xw_ref, dego_ref, y_ref, dinv_ref):
    deg = dego_ref[0, :N] + dego_ref[1, :N] + 1.0
    dinv = lax.rsqrt(deg)[:, None]
    dinv_ref[...] = dinv
    y_ref[...] = xw_ref[...] * dinv


def _dense2_body(za_ref, zb_ref, y1_ref, dinv_ref, b1_ref, w2_ref, y2_ref):
    dinv = dinv_ref[...]
    h = dinv * (za_ref[...] + zb_ref[...] + y1_ref[...]) + b1_ref[...][None, :]
    h = jnp.maximum(h, 0.0)
    y2 = jnp.dot(h, w2_ref[...], preferred_element_type=jnp.float32) * dinv
    y2_ref[...] = jnp.concatenate(
        [y2, jnp.zeros((N, D2 - NC), jnp.float32)], axis=1)


def _final_body(za_ref, zb_ref, y2_ref, dinv_ref, b2_ref, o_ref):
    t = dinv_ref[...] * (za_ref[...] + zb_ref[...] + y2_ref[...])
    logits = t[:, :NC] + b2_ref[...][None, :]
    m = jnp.max(logits, axis=1, keepdims=True)
    lse = jnp.log(jnp.sum(jnp.exp(logits - m), axis=1, keepdims=True)) + m
    o_ref[...] = logits - lse


def _tc(body, out_shape, *args):
    return pl.pallas_call(body, out_shape=out_shape)(*args)


def kernel(x, adj, W1, b1, W2, b2):
    src = adj[0].astype(jnp.int32)
    dst = adj[1].astype(jnp.int32)
    pad = E_PAD - E
    src3 = jnp.concatenate([src, jnp.zeros((pad,), jnp.int32)])
    src3 = src3.reshape(NWORK, NCHUNK, CHUNK)
    dst3 = jnp.concatenate([dst, jnp.full((pad,), TRASH, jnp.int32)])
    dst3 = dst3.reshape(NWORK, NCHUNK, CHUNK)

    zeros1 = jnp.zeros((N_PAD,), jnp.float32)
    zeros40 = jnp.zeros((N_PAD, NH), jnp.float32)
    zeros8 = jnp.zeros((N_PAD, D2), jnp.float32)
    ones = jnp.ones((CHUNK,), jnp.float32)

    dego = _deg_kernel(dst3, zeros1, ones)
    xw = _tc(_mm1_body, jax.ShapeDtypeStruct((N, NH), jnp.float32), x, W1)

    y1, dinv = pl.pallas_call(
        _scale_body,
        out_shape=[jax.ShapeDtypeStruct((N, NH), jnp.float32),
                   jax.ShapeDtypeStruct((N, 1), jnp.float32)],
    )(xw, dego)

    z1 = _spmm40(src3, dst3, y1, zeros40)
    y2 = _tc(_dense2_body, jax.ShapeDtypeStruct((N, D2), jnp.float32),
             z1[0, :N], z1[1, :N], y1, dinv, b1, W2)

    z2 = _spmm8(src3, dst3, y2, zeros8)
    out = _tc(_final_body, jax.ShapeDtypeStruct((N, NC), jnp.float32),
              z2[0, :N], z2[1, :N], y2, dinv, b2)
    return out


# speedup vs baseline: 48.3810x; 1.6030x over previous
import functools

import jax
import jax.numpy as jnp
from jax import lax
from jax.experimental import pallas as pl
from jax.experimental.pallas import tpu as pltpu
from jax.experimental.pallas import tpu_sc as plsc

N = 10000
E = 320000
NF = 128
NH = 40
NC = 7
D2 = 8

NWORK = 32
CHUNK = 128
NCHUNK = 80
E_PAD = NWORK * NCHUNK * CHUNK
N_PAD = 10240
TRASH = N
RPS = N_PAD // 16


def _mesh():
    return plsc.VectorSubcoreMesh(core_axis_name="c", subcore_axis_name="s")


_SC_PARAMS = pltpu.CompilerParams(use_tc_tiling_on_sc=False)


@functools.partial(
    pl.kernel,
    out_type=jax.ShapeDtypeStruct((2, N_PAD), jnp.float32),
    mesh=_mesh(),
    compiler_params=_SC_PARAMS,
    scratch_types=[
        pltpu.VMEM((NCHUNK, CHUNK), jnp.int32),
        pltpu.VMEM((CHUNK,), jnp.float32),
        pltpu.VMEM_SHARED((N_PAD,), jnp.float32),
    ],
)
def _deg_kernel(dst_hbm, zeros_hbm, ones_hbm, out_hbm, dst_v, ones_v, acc_sh):
    c = lax.axis_index("c")
    s = lax.axis_index("s")
    wid = c * 16 + s
    pltpu.sync_copy(dst_hbm.at[wid], dst_v)
    pltpu.sync_copy(ones_hbm, ones_v)
    sl = pl.ds(s * RPS, RPS)
    pltpu.sync_copy(zeros_hbm.at[sl], acc_sh.at[sl])
    plsc.subcore_barrier()

    def chunk(j, carry):
        pltpu.sync_copy(ones_v, acc_sh.at[dst_v.at[j]], add=True)
        return carry

    lax.fori_loop(0, NCHUNK, chunk, 0)
    plsc.subcore_barrier()
    pltpu.sync_copy(acc_sh.at[sl], out_hbm.at[c, sl])


NBUF = 4


def _make_spmm(d):
    @functools.partial(
        pl.kernel,
        out_type=jax.ShapeDtypeStruct((2, N_PAD, d), jnp.float32),
        mesh=_mesh(),
        compiler_params=_SC_PARAMS,
        scratch_types=[
            pltpu.VMEM((NCHUNK, CHUNK), jnp.int32),
            pltpu.VMEM((NCHUNK, CHUNK), jnp.int32),
            pltpu.VMEM((NBUF, CHUNK, d), jnp.float32),
            pltpu.VMEM_SHARED((N_PAD, d), jnp.float32),
            pltpu.VMEM_SHARED((N_PAD, d), jnp.float32),
            pltpu.SemaphoreType.DMA((NBUF,)),
        ],
    )
    def spmm(src_hbm, dst_hbm, y_hbm, zeros_hbm, out_hbm,
             src_v, dst_v, rows_v, acc_sh, y_sh, gsem):
        c = lax.axis_index("c")
        s = lax.axis_index("s")
        wid = c * 16 + s
        pltpu.sync_copy(src_hbm.at[wid], src_v)
        pltpu.sync_copy(dst_hbm.at[wid], dst_v)
        sl = pl.ds(s * RPS, RPS)
        pltpu.sync_copy(zeros_hbm.at[sl], acc_sh.at[sl])
        pltpu.sync_copy(y_hbm.at[sl], y_sh.at[sl])
        plsc.subcore_barrier()

        for b in range(NBUF):
            pltpu.make_async_copy(
                y_sh.at[src_v.at[b]], rows_v.at[b], gsem.at[b]).start()

        def group(i, carry):
            base = i * NBUF
            for b in range(NBUF):
                j = base + b
                pltpu.make_async_copy(
                    y_sh.at[src_v.at[j]], rows_v.at[b], gsem.at[b]).wait()
                pltpu.sync_copy(rows_v.at[b], acc_sh.at[dst_v.at[j]],
                                add=True)
                nxt = j + NBUF

                @pl.when(nxt < NCHUNK)
                def _():
                    pltpu.make_async_copy(
                        y_sh.at[src_v.at[nxt]], rows_v.at[b],
                        gsem.at[b]).start()
            return carry

        lax.fori_loop(0, NCHUNK // NBUF, group, 0)
        plsc.subcore_barrier()
        pltpu.sync_copy(acc_sh.at[sl], out_hbm.at[c, sl])

    return spmm


_spmm40 = _make_spmm(NH)
_spmm8 = _make_spmm(D2)


def _mm1_body(x_ref, w_ref, o_ref):
    o_ref[...] = jnp.dot(x_ref[...], w_ref[...],
                         preferred_element_type=jnp.float32)


def _scale_body(xw_ref, dego_ref, y_ref, dinv_ref):
    deg = dego_ref[0, :N] + dego_ref[1, :N] + 1.0
    dinv = lax.rsqrt(deg)[:, None]
    dinv_ref[...] = dinv
    y_ref[...] = jnp.concatenate(
        [xw_ref[...] * dinv, jnp.zeros((N_PAD - N, NH), jnp.float32)], axis=0)


def _dense2_body(za_ref, zb_ref, y1_ref, dinv_ref, b1_ref, w2_ref, y2_ref):
    dinv = dinv_ref[...]
    h = dinv * (za_ref[...] + zb_ref[...] + y1_ref[:N]) + b1_ref[...][None, :]
    h = jnp.maximum(h, 0.0)
    y2 = jnp.dot(h, w2_ref[...], preferred_element_type=jnp.float32) * dinv
    y2_ref[...] = jnp.pad(y2, ((0, N_PAD - N), (0, D2 - NC)))


def _final_body(za_ref, zb_ref, y2_ref, dinv_ref, b2_ref, o_ref):
    t = dinv_ref[...] * (za_ref[...] + zb_ref[...] + y2_ref[:N])
    logits = t[:, :NC] + b2_ref[...][None, :]
    m = jnp.max(logits, axis=1, keepdims=True)
    lse = jnp.log(jnp.sum(jnp.exp(logits - m), axis=1, keepdims=True)) + m
    o_ref[...] = logits - lse


def _tc(body, out_shape, *args):
    return pl.pallas_call(body, out_shape=out_shape)(*args)


def kernel(x, adj, W1, b1, W2, b2):
    src = adj[0].astype(jnp.int32)
    dst = adj[1].astype(jnp.int32)
    pad = E_PAD - E
    src3 = jnp.concatenate([src, jnp.zeros((pad,), jnp.int32)])
    src3 = src3.reshape(NWORK, NCHUNK, CHUNK)
    dst3 = jnp.concatenate([dst, jnp.full((pad,), TRASH, jnp.int32)])
    dst3 = dst3.reshape(NWORK, NCHUNK, CHUNK)

    zeros1 = jnp.zeros((N_PAD,), jnp.float32)
    zeros40 = jnp.zeros((N_PAD, NH), jnp.float32)
    zeros8 = jnp.zeros((N_PAD, D2), jnp.float32)
    ones = jnp.ones((CHUNK,), jnp.float32)

    dego = _deg_kernel(dst3, zeros1, ones)
    xw = _tc(_mm1_body, jax.ShapeDtypeStruct((N, NH), jnp.float32), x, W1)

    y1, dinv = pl.pallas_call(
        _scale_body,
        out_shape=[jax.ShapeDtypeStruct((N_PAD, NH), jnp.float32),
                   jax.ShapeDtypeStruct((N, 1), jnp.float32)],
    )(xw, dego)

    z1 = _spmm40(src3, dst3, y1, zeros40)
    y2 = _tc(_dense2_body, jax.ShapeDtypeStruct((N_PAD, D2), jnp.float32),
             z1[0, :N], z1[1, :N], y1, dinv, b1, W2)

    z2 = _spmm8(src3, dst3, y2, zeros8)
    out = _tc(_final_body, jax.ShapeDtypeStruct((N, NC), jnp.float32),
              z2[0, :N], z2[1, :N], y2, dinv, b2)
    return out


# speedup vs baseline: 54.5495x; 1.1275x over previous
import functools

import jax
import jax.numpy as jnp
from jax import lax
from jax.experimental import pallas as pl
from jax.experimental.pallas import tpu as pltpu
from jax.experimental.pallas import tpu_sc as plsc

N = 10000
E = 320000
NF = 128
NH = 40
NC = 7
D2 = 8

NWORK = 32
CHUNK = 128
NCHUNK = 80
E_PAD = NWORK * NCHUNK * CHUNK
N_PAD = 10240
TRASH = 10100
RPS = N_PAD // 16
NBUF = 4


def _mesh():
    return plsc.VectorSubcoreMesh(core_axis_name="c", subcore_axis_name="s")


_SC_PARAMS = pltpu.CompilerParams(use_tc_tiling_on_sc=False)


@functools.partial(
    pl.kernel,
    out_type=jax.ShapeDtypeStruct((2, N_PAD), jnp.float32),
    mesh=_mesh(),
    compiler_params=_SC_PARAMS,
    scratch_types=[
        pltpu.VMEM((NCHUNK, CHUNK), jnp.int32),
        pltpu.VMEM((CHUNK,), jnp.float32),
        pltpu.VMEM_SHARED((N_PAD,), jnp.float32),
        pltpu.SemaphoreType.DMA,
    ],
)
def _deg_kernel(adj_hbm, zeros_hbm, ones_hbm, out_hbm, dst_v, ones_v,
                acc_sh, sem):
    c = lax.axis_index("c")
    s = lax.axis_index("s")
    wid = c * 16 + s
    pltpu.sync_copy(adj_hbm.at[1, wid], dst_v)
    pltpu.sync_copy(ones_hbm, ones_v)
    sl = pl.ds(s * RPS, RPS)
    pltpu.sync_copy(zeros_hbm.at[sl], acc_sh.at[sl])
    plsc.subcore_barrier()

    def fire(j, carry):
        pltpu.async_copy(ones_v, acc_sh.at[dst_v.at[j]], sem, add=True)
        return carry

    lax.fori_loop(0, NCHUNK, fire, 0)

    def drain(j, carry):
        pltpu.make_async_copy(ones_v, acc_sh.at[dst_v.at[j]], sem).wait()
        return carry

    lax.fori_loop(0, NCHUNK, drain, 0)
    plsc.subcore_barrier()
    pltpu.sync_copy(acc_sh.at[sl], out_hbm.at[c, sl])


def _make_spmm(d):
    @functools.partial(
        pl.kernel,
        out_type=jax.ShapeDtypeStruct((2, N_PAD, d), jnp.float32),
        mesh=_mesh(),
        compiler_params=_SC_PARAMS,
        scratch_types=[
            pltpu.VMEM((NCHUNK, CHUNK), jnp.int32),
            pltpu.VMEM((NCHUNK, CHUNK), jnp.int32),
            pltpu.VMEM((NBUF, CHUNK, d), jnp.float32),
            pltpu.VMEM_SHARED((N_PAD, d), jnp.float32),
            pltpu.VMEM_SHARED((N_PAD, d), jnp.float32),
            pltpu.SemaphoreType.DMA((NBUF,)),
        ],
    )
    def spmm(adj_hbm, y_hbm, zeros_hbm, out_hbm,
             src_v, dst_v, rows_v, acc_sh, y_sh, gsem):
        c = lax.axis_index("c")
        s = lax.axis_index("s")
        wid = c * 16 + s
        pltpu.sync_copy(adj_hbm.at[0, wid], src_v)
        pltpu.sync_copy(adj_hbm.at[1, wid], dst_v)
        sl = pl.ds(s * RPS, RPS)
        pltpu.sync_copy(zeros_hbm.at[sl], acc_sh.at[sl])
        pltpu.sync_copy(y_hbm.at[sl], y_sh.at[sl])
        plsc.subcore_barrier()

        for b in range(NBUF):
            pltpu.make_async_copy(
                y_sh.at[src_v.at[b]], rows_v.at[b], gsem.at[b]).start()

        def group(i, carry):
            base = i * NBUF
            for b in range(NBUF):
                j = base + b
                pltpu.make_async_copy(
                    y_sh.at[src_v.at[j]], rows_v.at[b], gsem.at[b]).wait()
                pltpu.sync_copy(rows_v.at[b], acc_sh.at[dst_v.at[j]],
                                add=True)
                nxt = j + NBUF

                @pl.when(nxt < NCHUNK)
                def _():
                    pltpu.make_async_copy(
                        y_sh.at[src_v.at[nxt]], rows_v.at[b],
                        gsem.at[b]).start()
            return carry

        lax.fori_loop(0, NCHUNK // NBUF, group, 0)
        plsc.subcore_barrier()
        pltpu.sync_copy(acc_sh.at[sl], out_hbm.at[c, sl])

    return spmm


_spmm40 = _make_spmm(NH)
_spmm8 = _make_spmm(D2)


def _mm1_body(x_ref, dego_ref, w_ref, y_ref, dinv_ref):
    deg = dego_ref[0, :N] + dego_ref[1, :N] + 1.0
    dinv = lax.rsqrt(deg)[:, None]
    dinv_ref[...] = dinv
    y = jnp.dot(x_ref[...] * dinv, w_ref[...],
                preferred_element_type=jnp.float32)
    y_ref[...] = jnp.concatenate(
        [y, jnp.zeros((N_PAD - N, NH), jnp.float32)], axis=0)


def _dense2_body(z1_ref, y1_ref, dinv_ref, b1_ref, w2_ref, y2_ref):
    dinv = dinv_ref[...]
    h = dinv * (z1_ref[0, :N] + z1_ref[1, :N] + y1_ref[:N]) \
        + b1_ref[...][None, :]
    h = jnp.maximum(h, 0.0)
    y2 = jnp.dot(h, w2_ref[...], preferred_element_type=jnp.float32) * dinv
    y2_ref[...] = jnp.pad(y2, ((0, N_PAD - N), (0, D2 - NC)))


def _final_body(z2_ref, y2_ref, dinv_ref, b2_ref, o_ref):
    t = dinv_ref[...] * (z2_ref[0, :N] + z2_ref[1, :N] + y2_ref[:N])
    logits = t[:, :NC] + b2_ref[...][None, :]
    m = jnp.max(logits, axis=1, keepdims=True)
    lse = jnp.log(jnp.sum(jnp.exp(logits - m), axis=1, keepdims=True)) + m
    o_ref[...] = logits - lse


def _tc(body, out_shape, *args):
    return pl.pallas_call(body, out_shape=out_shape)(*args)


def kernel(x, adj, W1, b1, W2, b2):
    adj3 = jnp.pad(adj.astype(jnp.int32), ((0, 0), (0, E_PAD - E)),
                   constant_values=TRASH)
    adj3 = adj3.reshape(2, NWORK, NCHUNK, CHUNK)

    zeros1 = jnp.zeros((N_PAD,), jnp.float32)
    zeros40 = jnp.zeros((N_PAD, NH), jnp.float32)
    zeros8 = jnp.zeros((N_PAD, D2), jnp.float32)
    ones = jnp.ones((CHUNK,), jnp.float32)

    dego = _deg_kernel(adj3, zeros1, ones)
    y1, dinv = pl.pallas_call(
        _mm1_body,
        out_shape=[jax.ShapeDtypeStruct((N_PAD, NH), jnp.float32),
                   jax.ShapeDtypeStruct((N, 1), jnp.float32)],
    )(x, dego, W1)

    z1 = _spmm40(adj3, y1, zeros40)
    y2 = _tc(_dense2_body, jax.ShapeDtypeStruct((N_PAD, D2), jnp.float32),
             z1, y1, dinv, b1, W2)

    z2 = _spmm8(adj3, y2, zeros8)
    out = _tc(_final_body, jax.ShapeDtypeStruct((N, NC), jnp.float32),
              z2, y2, dinv, b2)
    return out


# speedup vs baseline: 54.8998x; 1.0064x over previous
import functools

import jax
import jax.numpy as jnp
from jax import lax
from jax.experimental import pallas as pl
from jax.experimental.pallas import tpu as pltpu
from jax.experimental.pallas import tpu_sc as plsc

N = 10000
E = 320000
NF = 128
NH = 40
NC = 7
D2 = 8

NWORK = 32
CHUNK = 128
NCHUNK = 80
E_PAD = NWORK * NCHUNK * CHUNK
N_PAD = 10240
TRASH = 10100
RPS = N_PAD // 16
NBUF = 4


def _mesh():
    return plsc.VectorSubcoreMesh(core_axis_name="c", subcore_axis_name="s")


_SC_PARAMS = pltpu.CompilerParams(use_tc_tiling_on_sc=False)


@functools.partial(
    pl.kernel,
    out_type=jax.ShapeDtypeStruct((2, N_PAD), jnp.float32),
    mesh=_mesh(),
    compiler_params=_SC_PARAMS,
    scratch_types=[
        pltpu.VMEM((NCHUNK, CHUNK), jnp.int32),
        pltpu.VMEM((CHUNK,), jnp.float32),
        pltpu.VMEM_SHARED((N_PAD,), jnp.float32),
        pltpu.SemaphoreType.DMA,
    ],
)
def _deg_kernel(adj_hbm, zeros_hbm, ones_hbm, out_hbm, dst_v, ones_v,
                acc_sh, sem):
    c = lax.axis_index("c")
    s = lax.axis_index("s")
    wid = c * 16 + s
    pltpu.sync_copy(adj_hbm.at[1, wid], dst_v)
    pltpu.sync_copy(ones_hbm, ones_v)
    sl = pl.ds(s * RPS, RPS)
    pltpu.sync_copy(zeros_hbm.at[sl], acc_sh.at[sl])
    plsc.subcore_barrier()

    def fire(j, carry):
        pltpu.async_copy(ones_v, acc_sh.at[dst_v.at[j]], sem, add=True)
        return carry

    lax.fori_loop(0, NCHUNK, fire, 0)

    def drain(j, carry):
        pltpu.make_async_copy(ones_v, acc_sh.at[dst_v.at[j]], sem).wait()
        return carry

    lax.fori_loop(0, NCHUNK, drain, 0)
    plsc.subcore_barrier()
    pltpu.sync_copy(acc_sh.at[sl], out_hbm.at[c, sl])


def _make_spmm(d):
    @functools.partial(
        pl.kernel,
        out_type=jax.ShapeDtypeStruct((2, N_PAD, d), jnp.float32),
        mesh=_mesh(),
        compiler_params=_SC_PARAMS,
        scratch_types=[
            pltpu.VMEM((NCHUNK, CHUNK), jnp.int32),
            pltpu.VMEM((NCHUNK, CHUNK), jnp.int32),
            pltpu.VMEM((NBUF, CHUNK, d), jnp.float32),
            pltpu.VMEM_SHARED((N_PAD, d), jnp.float32),
            pltpu.VMEM_SHARED((N_PAD, d), jnp.float32),
            pltpu.SemaphoreType.DMA((NBUF,)),
        ],
    )
    def spmm(adj_hbm, y_hbm, zeros_hbm, out_hbm,
             src_v, dst_v, rows_v, acc_sh, y_sh, gsem):
        c = lax.axis_index("c")
        s = lax.axis_index("s")
        wid = c * 16 + s
        pltpu.sync_copy(adj_hbm.at[0, wid], src_v)
        pltpu.sync_copy(adj_hbm.at[1, wid], dst_v)
        sl = pl.ds(s * RPS, RPS)
        pltpu.sync_copy(zeros_hbm.at[sl], acc_sh.at[sl])
        pltpu.sync_copy(y_hbm.at[sl], y_sh.at[sl])
        plsc.subcore_barrier()

        for b in range(NBUF):
            pltpu.make_async_copy(
                y_sh.at[src_v.at[b]], rows_v.at[b], gsem.at[b]).start()

        def group(i, carry):
            base = i * NBUF
            for b in range(NBUF):
                j = base + b
                pltpu.make_async_copy(
                    y_sh.at[src_v.at[j]], rows_v.at[b], gsem.at[b]).wait()
                pltpu.sync_copy(rows_v.at[b], acc_sh.at[dst_v.at[j]],
                                add=True)
                nxt = j + NBUF

                @pl.when(nxt < NCHUNK)
                def _():
                    pltpu.make_async_copy(
                        y_sh.at[src_v.at[nxt]], rows_v.at[b],
                        gsem.at[b]).start()
            return carry

        lax.fori_loop(0, NCHUNK // NBUF, group, 0)
        plsc.subcore_barrier()
        pltpu.sync_copy(acc_sh.at[sl], out_hbm.at[c, sl])

    return spmm


_spmm40 = _make_spmm(NH)
_spmm8 = _make_spmm(D2)


BR = 1280
GRID = N_PAD // BR


def _dinv_of(dego_ref):
    deg = dego_ref[0] + dego_ref[1] + 1.0
    return lax.rsqrt(deg)[:, None]


def _mm1_body(x_ref, dego_ref, w_ref, y_ref):
    y_ref[...] = jnp.dot(x_ref[...] * _dinv_of(dego_ref), w_ref[...],
                         preferred_element_type=jnp.float32)


def _dense2_body(z1_ref, y1_ref, dego_ref, b1_ref, w2_ref, y2_ref):
    dinv = _dinv_of(dego_ref)
    h = dinv * (z1_ref[0] + z1_ref[1] + y1_ref[...]) + b1_ref[...][None, :]
    h = jnp.maximum(h, 0.0)
    y2 = jnp.dot(h, w2_ref[...], preferred_element_type=jnp.float32) * dinv
    y2_ref[...] = jnp.pad(y2, ((0, 0), (0, D2 - NC)))


def _final_body(z2_ref, y2_ref, dego_ref, b2_ref, o_ref):
    t = _dinv_of(dego_ref) * (z2_ref[0] + z2_ref[1] + y2_ref[...])
    logits = t[:, :NC] + b2_ref[...][None, :]
    m = jnp.max(logits, axis=1, keepdims=True)
    lse = jnp.log(jnp.sum(jnp.exp(logits - m), axis=1, keepdims=True)) + m
    o_ref[...] = logits - lse


def _rows(d):
    return pl.BlockSpec((BR, d), lambda i: (i, 0))


def _rows2(d):
    return pl.BlockSpec((2, BR, d), lambda i: (0, i, 0))


_DEGO_SPEC = pl.BlockSpec((2, BR), lambda i: (0, i))


def _full(shape):
    return pl.BlockSpec(shape, lambda i: tuple(0 for _ in shape))


def kernel(x, adj, W1, b1, W2, b2):
    adj3 = jnp.pad(adj.astype(jnp.int32), ((0, 0), (0, E_PAD - E)),
                   constant_values=TRASH)
    adj3 = adj3.reshape(2, NWORK, NCHUNK, CHUNK)

    zeros1 = jnp.zeros((N_PAD,), jnp.float32)
    zeros40 = jnp.zeros((N_PAD, NH), jnp.float32)
    zeros8 = jnp.zeros((N_PAD, D2), jnp.float32)
    ones = jnp.ones((CHUNK,), jnp.float32)

    dego = _deg_kernel(adj3, zeros1, ones)
    y1 = pl.pallas_call(
        _mm1_body,
        grid=(GRID,),
        in_specs=[_rows(NF), _DEGO_SPEC, _full((NF, NH))],
        out_specs=_rows(NH),
        out_shape=jax.ShapeDtypeStruct((N_PAD, NH), jnp.float32),
    )(x, dego, W1)

    z1 = _spmm40(adj3, y1, zeros40)
    y2 = pl.pallas_call(
        _dense2_body,
        grid=(GRID,),
        in_specs=[_rows2(NH), _rows(NH), _DEGO_SPEC, _full((NH,)),
                  _full((NH, NC))],
        out_specs=_rows(D2),
        out_shape=jax.ShapeDtypeStruct((N_PAD, D2), jnp.float32),
    )(z1, y1, dego, b1, W2)

    z2 = _spmm8(adj3, y2, zeros8)
    out = pl.pallas_call(
        _final_body,
        grid=(GRID,),
        in_specs=[_rows2(D2), _rows(D2), _DEGO_SPEC, _full((NC,))],
        out_specs=_rows(NC),
        out_shape=jax.ShapeDtypeStruct((N, NC), jnp.float32),
    )(z2, y2, dego, b2)
    return out


# speedup vs baseline: 62.8301x; 1.1445x over previous
import functools

import jax
import jax.numpy as jnp
from jax import lax
from jax.experimental import pallas as pl
from jax.experimental.pallas import tpu as pltpu
from jax.experimental.pallas import tpu_sc as plsc

N = 10000
E = 320000
NF = 128
NH = 40
NC = 7
D2 = 8
LW = 128

NWORK = 32
CHUNK = 128
NCHUNK = 80
E_PAD = NWORK * NCHUNK * CHUNK
N_PAD = 10240
TRASH = 10100
RPS = N_PAD // 16
NBUF = 4


def _mesh():
    return plsc.VectorSubcoreMesh(core_axis_name="c", subcore_axis_name="s")


_SC_PARAMS = pltpu.CompilerParams(use_tc_tiling_on_sc=False)


@functools.partial(
    pl.kernel,
    out_type=jax.ShapeDtypeStruct((2, N_PAD), jnp.float32),
    mesh=_mesh(),
    compiler_params=_SC_PARAMS,
    scratch_types=[
        pltpu.VMEM((NCHUNK, CHUNK), jnp.int32),
        pltpu.VMEM((CHUNK,), jnp.float32),
        pltpu.VMEM_SHARED((N_PAD,), jnp.float32),
        pltpu.SemaphoreType.DMA,
    ],
)
def _deg_kernel(adj_hbm, zeros_hbm, ones_hbm, out_hbm, dst_v, ones_v,
                acc_sh, sem):
    c = lax.axis_index("c")
    s = lax.axis_index("s")
    wid = c * 16 + s
    pltpu.sync_copy(adj_hbm.at[1, wid], dst_v)
    pltpu.sync_copy(ones_hbm, ones_v)
    sl = pl.ds(s * RPS, RPS)
    pltpu.sync_copy(zeros_hbm.at[sl], acc_sh.at[sl])
    plsc.subcore_barrier()

    def fire(j, carry):
        pltpu.async_copy(ones_v, acc_sh.at[dst_v.at[j]], sem, add=True)
        return carry

    lax.fori_loop(0, NCHUNK, fire, 0)

    def drain(j, carry):
        pltpu.make_async_copy(ones_v, acc_sh.at[dst_v.at[j]], sem).wait()
        return carry

    lax.fori_loop(0, NCHUNK, drain, 0)
    plsc.subcore_barrier()
    pltpu.sync_copy(acc_sh.at[sl], out_hbm.at[c, sl])


def _make_spmm(d):
    @functools.partial(
        pl.kernel,
        out_type=jax.ShapeDtypeStruct((N_PAD, LW), jnp.float32),
        mesh=_mesh(),
        compiler_params=_SC_PARAMS,
        scratch_types=[
            pltpu.VMEM((NCHUNK, CHUNK), jnp.int32),
            pltpu.VMEM((NCHUNK, CHUNK), jnp.int32),
            pltpu.VMEM((NBUF, CHUNK, d), jnp.float32),
            pltpu.VMEM_SHARED((N_PAD, d), jnp.float32),
            pltpu.VMEM_SHARED((N_PAD, d), jnp.float32),
            pltpu.SemaphoreType.DMA((NBUF,)),
        ],
    )
    def spmm(adj_hbm, y_hbm, zeros_hbm, out_hbm,
             src_v, dst_v, rows_v, acc_sh, y_sh, gsem):
        c = lax.axis_index("c")
        s = lax.axis_index("s")
        wid = c * 16 + s
        pltpu.sync_copy(adj_hbm.at[0, wid], src_v)
        pltpu.sync_copy(adj_hbm.at[1, wid], dst_v)
        sl = pl.ds(s * RPS, RPS)
        pltpu.sync_copy(zeros_hbm.at[sl], acc_sh.at[sl])
        pltpu.sync_copy(y_hbm.at[sl, pl.ds(0, d)], y_sh.at[sl])
        plsc.subcore_barrier()

        for b in range(NBUF):
            pltpu.make_async_copy(
                y_sh.at[src_v.at[b]], rows_v.at[b], gsem.at[b]).start()

        def group(i, carry):
            base = i * NBUF
            for b in range(NBUF):
                j = base + b
                pltpu.make_async_copy(
                    y_sh.at[src_v.at[j]], rows_v.at[b], gsem.at[b]).wait()
                pltpu.sync_copy(rows_v.at[b], acc_sh.at[dst_v.at[j]],
                                add=True)
                nxt = j + NBUF

                @pl.when(nxt < NCHUNK)
                def _():
                    pltpu.make_async_copy(
                        y_sh.at[src_v.at[nxt]], rows_v.at[b],
                        gsem.at[b]).start()
            return carry

        lax.fori_loop(0, NCHUNK // NBUF, group, 0)
        plsc.subcore_barrier()
        pltpu.sync_copy(acc_sh.at[sl], out_hbm.at[sl, pl.ds(c * d, d)])

    return spmm


_spmm40 = _make_spmm(NH)
_spmm8 = _make_spmm(D2)


BR = 1280
GRID = N_PAD // BR


def _dinv_of(dego_ref):
    deg = dego_ref[0] + dego_ref[1] + 1.0
    return lax.rsqrt(deg)[:, None]


def _mm1_body(x_ref, dego_ref, w_ref, y_ref):
    y = jnp.dot(x_ref[...] * _dinv_of(dego_ref), w_ref[...],
                preferred_element_type=jnp.float32)
    y_ref[...] = jnp.pad(y, ((0, 0), (0, LW - NH)))


def _dense2_body(z1_ref, y1_ref, dego_ref, b1_ref, w2_ref, y2_ref):
    dinv = _dinv_of(dego_ref)
    z1 = z1_ref[:, :NH] + z1_ref[:, NH:2 * NH]
    h = dinv * (z1 + y1_ref[:, :NH]) + b1_ref[...][None, :]
    h = jnp.maximum(h, 0.0)
    y2 = jnp.dot(h, w2_ref[...], preferred_element_type=jnp.float32) * dinv
    y2_ref[...] = jnp.pad(y2, ((0, 0), (0, LW - NC)))


def _final_body(z2_ref, y2_ref, dego_ref, b2_ref, o_ref):
    z2 = z2_ref[:, :D2] + z2_ref[:, D2:2 * D2]
    t = _dinv_of(dego_ref) * (z2 + y2_ref[:, :D2])
    logits = t[:, :NC] + b2_ref[...][None, :]
    m = jnp.max(logits, axis=1, keepdims=True)
    lse = jnp.log(jnp.sum(jnp.exp(logits - m), axis=1, keepdims=True)) + m
    o_ref[...] = logits - lse


def _rows(d):
    return pl.BlockSpec((BR, d), lambda i: (i, 0))


_DEGO_SPEC = pl.BlockSpec((2, BR), lambda i: (0, i))


def _full(shape):
    return pl.BlockSpec(shape, lambda i: tuple(0 for _ in shape))


def kernel(x, adj, W1, b1, W2, b2):
    adj3 = jnp.pad(adj.astype(jnp.int32), ((0, 0), (0, E_PAD - E)),
                   constant_values=TRASH)
    adj3 = adj3.reshape(2, NWORK, NCHUNK, CHUNK)

    zeros1 = jnp.zeros((N_PAD,), jnp.float32)
    zeros40 = jnp.zeros((N_PAD, NH), jnp.float32)
    zeros8 = jnp.zeros((N_PAD, D2), jnp.float32)
    ones = jnp.ones((CHUNK,), jnp.float32)

    dego = _deg_kernel(adj3, zeros1, ones)
    y1 = pl.pallas_call(
        _mm1_body,
        grid=(GRID,),
        in_specs=[_rows(NF), _DEGO_SPEC, _full((NF, NH))],
        out_specs=_rows(LW),
        out_shape=jax.ShapeDtypeStruct((N_PAD, LW), jnp.float32),
    )(x, dego, W1)

    z1 = _spmm40(adj3, y1, zeros40)
    y2 = pl.pallas_call(
        _dense2_body,
        grid=(GRID,),
        in_specs=[_rows(LW), _rows(LW), _DEGO_SPEC, _full((NH,)),
                  _full((NH, NC))],
        out_specs=_rows(LW),
        out_shape=jax.ShapeDtypeStruct((N_PAD, LW), jnp.float32),
    )(z1, y1, dego, b1, W2)

    z2 = _spmm8(adj3, y2, zeros8)
    out = pl.pallas_call(
        _final_body,
        grid=(GRID,),
        in_specs=[_rows(LW), _rows(LW), _DEGO_SPEC, _full((NC,))],
        out_specs=_rows(NC),
        out_shape=jax.ShapeDtypeStruct((N, NC), jnp.float32),
    )(z2, y2, dego, b2)
    return out


# speedup vs baseline: 62.8375x; 1.0001x over previous
import functools

import jax
import jax.numpy as jnp
from jax import lax
from jax.experimental import pallas as pl
from jax.experimental.pallas import tpu as pltpu
from jax.experimental.pallas import tpu_sc as plsc

N = 10000
E = 320000
NF = 128
NH = 40
NC = 7
D2 = 8
LW = 128

NWORK = 32
CHUNK = 128
NCHUNK = 80
E_PAD = NWORK * NCHUNK * CHUNK
N_PAD = 10240
TRASH = 10100
RPS = N_PAD // 16
NBUF = 4


def _mesh():
    return plsc.VectorSubcoreMesh(core_axis_name="c", subcore_axis_name="s")


_SC_PARAMS = pltpu.CompilerParams(use_tc_tiling_on_sc=False)


@functools.partial(
    pl.kernel,
    out_type=jax.ShapeDtypeStruct((2, N_PAD), jnp.float32),
    mesh=_mesh(),
    compiler_params=_SC_PARAMS,
    scratch_types=[
        pltpu.VMEM((NCHUNK, CHUNK), jnp.int32),
        pltpu.VMEM((CHUNK,), jnp.float32),
        pltpu.VMEM_SHARED((N_PAD,), jnp.float32),
        pltpu.SemaphoreType.DMA,
    ],
)
def _deg_kernel(adj_hbm, zeros_hbm, ones_hbm, out_hbm, dst_v, ones_v,
                acc_sh, sem):
    c = lax.axis_index("c")
    s = lax.axis_index("s")
    wid = c * 16 + s
    pltpu.sync_copy(adj_hbm.at[1, wid], dst_v)
    pltpu.sync_copy(ones_hbm, ones_v)
    sl = pl.ds(s * RPS, RPS)
    pltpu.sync_copy(zeros_hbm.at[sl], acc_sh.at[sl])
    plsc.subcore_barrier()

    def fire(j, carry):
        pltpu.async_copy(ones_v, acc_sh.at[dst_v.at[j]], sem, add=True)
        return carry

    lax.fori_loop(0, NCHUNK, fire, 0)

    def drain(j, carry):
        pltpu.make_async_copy(ones_v, acc_sh.at[dst_v.at[j]], sem).wait()
        return carry

    lax.fori_loop(0, NCHUNK, drain, 0)
    plsc.subcore_barrier()
    pltpu.sync_copy(acc_sh.at[sl], out_hbm.at[c, sl])


def _make_spmm(d, nbuf=NBUF):
    @functools.partial(
        pl.kernel,
        out_type=jax.ShapeDtypeStruct((N_PAD, LW), jnp.float32),
        mesh=_mesh(),
        compiler_params=_SC_PARAMS,
        scratch_types=[
            pltpu.VMEM((NCHUNK, CHUNK), jnp.int32),
            pltpu.VMEM((NCHUNK, CHUNK), jnp.int32),
            pltpu.VMEM((nbuf, CHUNK, d), jnp.float32),
            pltpu.VMEM_SHARED((N_PAD, d), jnp.float32),
            pltpu.VMEM_SHARED((N_PAD, d), jnp.float32),
            pltpu.SemaphoreType.DMA((nbuf,)),
        ],
    )
    def spmm(adj_hbm, y_hbm, zeros_hbm, out_hbm,
             src_v, dst_v, rows_v, acc_sh, y_sh, gsem):
        c = lax.axis_index("c")
        s = lax.axis_index("s")
        wid = c * 16 + s
        pltpu.sync_copy(adj_hbm.at[0, wid], src_v)
        pltpu.sync_copy(adj_hbm.at[1, wid], dst_v)
        sl = pl.ds(s * RPS, RPS)
        pltpu.sync_copy(zeros_hbm.at[sl], acc_sh.at[sl])
        pltpu.sync_copy(y_hbm.at[sl, pl.ds(0, d)], y_sh.at[sl])
        plsc.subcore_barrier()

        for b in range(nbuf):
            pltpu.make_async_copy(
                y_sh.at[src_v.at[b]], rows_v.at[b], gsem.at[b]).start()

        def group(i, carry):
            base = i * nbuf
            for b in range(nbuf):
                j = base + b
                pltpu.make_async_copy(
                    y_sh.at[src_v.at[j]], rows_v.at[b], gsem.at[b]).wait()
                pltpu.sync_copy(rows_v.at[b], acc_sh.at[dst_v.at[j]],
                                add=True)
                nxt = j + nbuf

                @pl.when(nxt < NCHUNK)
                def _():
                    pltpu.make_async_copy(
                        y_sh.at[src_v.at[nxt]], rows_v.at[b],
                        gsem.at[b]).start()
            return carry

        lax.fori_loop(0, NCHUNK // nbuf, group, 0)
        plsc.subcore_barrier()
        pltpu.sync_copy(acc_sh.at[sl], out_hbm.at[sl, pl.ds(c * d, d)])

    return spmm


_spmm40 = _make_spmm(NH)
_spmm8 = _make_spmm(D2, nbuf=8)


BR = 1280
GRID = N_PAD // BR


def _dinv_of(dego_ref):
    deg = dego_ref[0] + dego_ref[1] + 1.0
    return lax.rsqrt(deg)[:, None]


def _mm1_body(x_ref, dego_ref, w_ref, y_ref):
    y = jnp.dot(x_ref[...] * _dinv_of(dego_ref), w_ref[...],
                preferred_element_type=jnp.float32)
    y_ref[...] = jnp.pad(y, ((0, 0), (0, LW - NH)))


def _dense2_body(z1_ref, y1_ref, dego_ref, b1_ref, w2_ref, y2_ref):
    dinv = _dinv_of(dego_ref)
    z1 = z1_ref[:, :NH] + z1_ref[:, NH:2 * NH]
    h = dinv * (z1 + y1_ref[:, :NH]) + b1_ref[...][None, :]
    h = jnp.maximum(h, 0.0)
    y2 = jnp.dot(h, w2_ref[...], preferred_element_type=jnp.float32) * dinv
    y2_ref[...] = jnp.pad(y2, ((0, 0), (0, LW - NC)))


def _final_body(z2_ref, y2_ref, dego_ref, b2_ref, o_ref):
    z2 = z2_ref[:, :D2] + z2_ref[:, D2:2 * D2]
    t = _dinv_of(dego_ref) * (z2 + y2_ref[:, :D2])
    logits = t[:, :NC] + b2_ref[...][None, :]
    m = jnp.max(logits, axis=1, keepdims=True)
    lse = jnp.log(jnp.sum(jnp.exp(logits - m), axis=1, keepdims=True)) + m
    o_ref[...] = logits - lse


def _rows(d):
    return pl.BlockSpec((BR, d), lambda i: (i, 0))


_DEGO_SPEC = pl.BlockSpec((2, BR), lambda i: (0, i))


def _full(shape):
    return pl.BlockSpec(shape, lambda i: tuple(0 for _ in shape))


def kernel(x, adj, W1, b1, W2, b2):
    adj3 = jnp.pad(adj.astype(jnp.int32), ((0, 0), (0, E_PAD - E)),
                   constant_values=TRASH)
    adj3 = adj3.reshape(2, NWORK, NCHUNK, CHUNK)

    zeros1 = jnp.zeros((N_PAD,), jnp.float32)
    zeros40 = jnp.zeros((N_PAD, NH), jnp.float32)
    zeros8 = jnp.zeros((N_PAD, D2), jnp.float32)
    ones = jnp.ones((CHUNK,), jnp.float32)

    dego = _deg_kernel(adj3, zeros1, ones)
    y1 = pl.pallas_call(
        _mm1_body,
        grid=(GRID,),
        in_specs=[_rows(NF), _DEGO_SPEC, _full((NF, NH))],
        out_specs=_rows(LW),
        out_shape=jax.ShapeDtypeStruct((N_PAD, LW), jnp.float32),
    )(x, dego, W1)

    z1 = _spmm40(adj3, y1, zeros40)
    y2 = pl.pallas_call(
        _dense2_body,
        grid=(GRID,),
        in_specs=[_rows(LW), _rows(LW), _DEGO_SPEC, _full((NH,)),
                  _full((NH, NC))],
        out_specs=_rows(LW),
        out_shape=jax.ShapeDtypeStruct((N_PAD, LW), jnp.float32),
    )(z1, y1, dego, b1, W2)

    z2 = _spmm8(adj3, y2, zeros8)
    out = pl.pallas_call(
        _final_body,
        grid=(GRID,),
        in_specs=[_rows(LW), _rows(LW), _DEGO_SPEC, _full((NC,))],
        out_specs=_rows(NC),
        out_shape=jax.ShapeDtypeStruct((N, NC), jnp.float32),
    )(z2, y2, dego, b2)
    return out
